# baseline (device time: 13348 ns/iter reference)
import jax
import jax.numpy as jnp
from jax import lax
from jax.experimental import pallas as pl
from jax.experimental.pallas import tpu as pltpu

N_Y = 4


def kernel(x):
    _, m, n_total = x.shape
    n = n_total // N_Y
    h = n // 2

    def body(
        x_ref,
        out_ref,
        p1s_r,
        p1s_l,
        p1r_l,
        p1r_r,
        p2s_r,
        p2s_l,
        p2r_l,
        p2r_r,
        send_sems,
        recv_sems,
    ):
        my_x = lax.axis_index("x")
        my_y = lax.axis_index("y")
        my_z = lax.axis_index("z")

        right = (my_y + 1) % N_Y
        left = (my_y + 3) % N_Y
        far = (my_y + 2) % N_Y

        barrier_sem = pltpu.get_barrier_semaphore()
        for nbr in (right, left):
            pl.semaphore_signal(
                barrier_sem,
                inc=1,
                device_id=(my_x, nbr, my_z),
                device_id_type=pl.DeviceIdType.MESH,
            )

        p1s_r[:, :] = x_ref[0, :, pl.ds(far * n, h)].astype(jnp.bfloat16)
        p1s_l[:, :] = x_ref[0, :, pl.ds(far * n + h, h)].astype(jnp.bfloat16)
        p2s_r[:, :] = x_ref[0, :, pl.ds(right * n, n)].astype(jnp.bfloat16)
        p2s_l[:, :] = x_ref[0, :, pl.ds(left * n, n)].astype(jnp.bfloat16)

        pl.semaphore_wait(barrier_sem, 2)

        def rdma(src, dst, i, peer):
            return pltpu.make_async_remote_copy(
                src_ref=src,
                dst_ref=dst,
                send_sem=send_sems.at[i],
                recv_sem=recv_sems.at[i],
                device_id=(my_x, peer, my_z),
                device_id_type=pl.DeviceIdType.MESH,
            )

        d0 = rdma(p1s_r, p1r_l, 0, right)
        d1 = rdma(p1s_l, p1r_r, 1, left)
        d2 = rdma(p2s_r, p2r_l, 2, right)
        d3 = rdma(p2s_l, p2r_r, 3, left)

        d0.start()
        d1.start()

        d0.wait_recv()
        p2s_r[:, 0:h] = p2s_r[:, 0:h] + p1r_l[:, :]
        d2.start()

        d1.wait_recv()
        p2s_l[:, h:n] = p2s_l[:, h:n] + p1r_r[:, :]
        d3.start()

        acc = x_ref[0, :, pl.ds(my_y * n, n)]
        d2.wait_recv()
        acc = acc + p2r_l[:, :].astype(jnp.float32)
        d3.wait_recv()
        acc = acc + p2r_r[:, :].astype(jnp.float32)
        out_ref[:, :] = acc

        d0.wait_send()
        d1.wait_send()
        d2.wait_send()
        d3.wait_send()

    return pl.pallas_call(
        body,
        out_shape=jax.ShapeDtypeStruct((m, n), jnp.float32),
        in_specs=[pl.BlockSpec(memory_space=pltpu.VMEM)],
        out_specs=pl.BlockSpec(memory_space=pltpu.VMEM),
        scratch_shapes=[
            pltpu.VMEM((m, h), jnp.bfloat16),
            pltpu.VMEM((m, h), jnp.bfloat16),
            pltpu.VMEM((m, h), jnp.bfloat16),
            pltpu.VMEM((m, h), jnp.bfloat16),
            pltpu.VMEM((m, n), jnp.bfloat16),
            pltpu.VMEM((m, n), jnp.bfloat16),
            pltpu.VMEM((m, n), jnp.bfloat16),
            pltpu.VMEM((m, n), jnp.bfloat16),
            pltpu.SemaphoreType.DMA((4,)),
            pltpu.SemaphoreType.DMA((4,)),
        ],
        compiler_params=pltpu.CompilerParams(collective_id=0),
    )(x)


# device time: 11783 ns/iter; 1.1328x vs baseline; 1.1328x over previous
import jax
import jax.numpy as jnp
from jax import lax
from jax.experimental import pallas as pl
from jax.experimental.pallas import tpu as pltpu

N_Y = 4


def kernel(x):
    _, m, n_total = x.shape
    n = n_total // N_Y

    def body(x_ref, out_ref, send_buf, recv_buf, send_sems, recv_sems):
        my_x = lax.axis_index("x")
        my_y = lax.axis_index("y")
        my_z = lax.axis_index("z")

        barrier_sem = pltpu.get_barrier_semaphore()
        for d in range(1, N_Y):
            peer = (my_y + d) % N_Y
            pl.semaphore_signal(
                barrier_sem,
                inc=1,
                device_id=(my_x, peer, my_z),
                device_id_type=pl.DeviceIdType.MESH,
            )

        for d in range(1, N_Y):
            peer = (my_y + d) % N_Y
            send_buf[d - 1, :, :] = x_ref[0, :, pl.ds(peer * n, n)].astype(
                jnp.bfloat16
            )

        pl.semaphore_wait(barrier_sem, N_Y - 1)

        rdmas = []
        for d in range(1, N_Y):
            peer = (my_y + d) % N_Y
            rdma = pltpu.make_async_remote_copy(
                src_ref=send_buf.at[d - 1],
                dst_ref=recv_buf.at[d - 1],
                send_sem=send_sems.at[d - 1],
                recv_sem=recv_sems.at[d - 1],
                device_id=(my_x, peer, my_z),
                device_id_type=pl.DeviceIdType.MESH,
            )
            rdma.start()
            rdmas.append(rdma)

        acc = x_ref[0, :, pl.ds(my_y * n, n)]
        for d in range(1, N_Y):
            rdmas[d - 1].wait_recv()
            acc = acc + recv_buf[d - 1, :, :].astype(jnp.float32)
        out_ref[:, :] = acc

        for d in range(1, N_Y):
            rdmas[d - 1].wait_send()

    return pl.pallas_call(
        body,
        out_shape=jax.ShapeDtypeStruct((m, n), jnp.float32),
        in_specs=[pl.BlockSpec(memory_space=pltpu.VMEM)],
        out_specs=pl.BlockSpec(memory_space=pltpu.VMEM),
        scratch_shapes=[
            pltpu.VMEM((N_Y - 1, m, n), jnp.bfloat16),
            pltpu.VMEM((N_Y - 1, m, n), jnp.bfloat16),
            pltpu.SemaphoreType.DMA((N_Y - 1,)),
            pltpu.SemaphoreType.DMA((N_Y - 1,)),
        ],
        compiler_params=pltpu.CompilerParams(collective_id=0),
    )(x)
